# baseline (speedup 1.0000x reference)
import functools

import jax
import jax.numpy as jnp
from jax import lax
from jax.experimental import pallas as pl
from jax.experimental.pallas import tpu as pltpu
from jax.experimental.pallas import tpu_sc as plsc

N = 2048
D = 768
E = 8
K = 2
F = 4 * D
BLK = 128
G = (N * K) // BLK + E
NB = G * BLK

NC, NS = 2, 16
NW = NC * NS
TPW = N // NW
LANES = 16
VPR = D // LANES


def _gate_route_body(x_ref, gw_ref, gb_ref, tw_ref, pos_ref, be_ref):
    xv = x_ref[...]
    logits = jnp.dot(xv, gw_ref[...], preferred_element_type=jnp.float32)
    logits = logits + gb_ref[...]
    m = jnp.max(logits, axis=1, keepdims=True)
    ex = jnp.exp(logits - m)
    p = ex / jnp.sum(ex, axis=1, keepdims=True)

    col = lax.broadcasted_iota(jnp.int32, (N, E), 1)
    m0 = jnp.max(p, axis=1, keepdims=True)
    i0 = jnp.min(jnp.where(p == m0, col, E), axis=1, keepdims=True)
    oh0 = (col == i0)
    p1 = jnp.where(oh0, -jnp.inf, p)
    m1 = jnp.max(p1, axis=1, keepdims=True)
    i1 = jnp.min(jnp.where(p1 == m1, col, E), axis=1, keepdims=True)
    oh1 = (col == i1)
    z128 = jnp.zeros((N, 128), jnp.float32)
    tw_ref[0:N, :] = m0 + z128
    tw_ref[N:2 * N, :] = m1 + z128

    ohf = (oh0 | oh1).astype(jnp.float32)
    c = ohf
    sh = 1
    while sh < N:
        c = c + jnp.concatenate([jnp.zeros((sh, E), jnp.float32), c[: N - sh]], axis=0)
        sh *= 2
    excl = c - ohf
    counts = jnp.sum(ohf, axis=0, keepdims=True)
    ci = counts.astype(jnp.int32)
    pc = ((ci + BLK - 1) // BLK) * BLK
    inc = pc
    sh = 1
    while sh < E:
        inc = inc + jnp.concatenate(
            [jnp.zeros((1, sh), jnp.int32), inc[:, : E - sh]], axis=1)
        sh *= 2
    off = (inc - pc).astype(jnp.float32)

    oh0f = oh0.astype(jnp.float32)
    oh1f = oh1.astype(jnp.float32)
    r0 = jnp.sum(excl * oh0f, axis=1, keepdims=True)
    r1 = jnp.sum(excl * oh1f, axis=1, keepdims=True)
    s0 = jnp.sum(off * oh0f, axis=1, keepdims=True)
    s1 = jnp.sum(off * oh1f, axis=1, keepdims=True)
    pos_ref[...] = jnp.concatenate([s0 + r0, s1 + r1], axis=1).astype(jnp.int32)

    bi = lax.broadcasted_iota(jnp.int32, (1, G), 1) * BLK
    acc = jnp.zeros((1, G), jnp.int32)
    offi = (inc - pc)
    for e in range(E):
        acc = acc + jnp.where(bi >= offi[:, e:e + 1], 1, 0)
    be_ref[...] = acc - 1


def _ffn_body(be_ref, x_ref, w1_ref, b1_ref, w2_ref, b2_ref, ws_ref, y_ref):
    del be_ref
    xb = x_ref[...].astype(jnp.bfloat16)
    h = jnp.dot(xb, w1_ref[0], preferred_element_type=jnp.float32)
    h = jnp.maximum(h + b1_ref[0], 0.0).astype(jnp.bfloat16)
    y = jnp.dot(h, w2_ref[0], preferred_element_type=jnp.float32) + b2_ref[0]
    y_ref[...] = y * ws_ref[:, :1]


def _grouped_ffn(be, xs, W1, b1, W2, b2, ws):
    grid_spec = pltpu.PrefetchScalarGridSpec(
        num_scalar_prefetch=1,
        grid=(G,),
        in_specs=[
            pl.BlockSpec((BLK, D), lambda g, be: (g, 0)),
            pl.BlockSpec((1, D, F), lambda g, be: (be[g], 0, 0)),
            pl.BlockSpec((1, 1, F), lambda g, be: (be[g], 0, 0)),
            pl.BlockSpec((1, F, D), lambda g, be: (be[g], 0, 0)),
            pl.BlockSpec((1, 1, D), lambda g, be: (be[g], 0, 0)),
            pl.BlockSpec((BLK, 128), lambda g, be: (g, 0)),
        ],
        out_specs=pl.BlockSpec((BLK, D), lambda g, be: (g, 0)),
    )
    return pl.pallas_call(
        _ffn_body,
        grid_spec=grid_spec,
        out_shape=jax.ShapeDtypeStruct((NB, D), jnp.float32),
        compiler_params=pltpu.CompilerParams(
            dimension_semantics=("arbitrary",)),
    )(be, xs, W1.astype(jnp.bfloat16), b1.reshape(E, 1, F),
      W2.astype(jnp.bfloat16), b2.reshape(E, 1, D), ws)


@functools.cache
def _sc_kernels():
    mesh = plsc.VectorSubcoreMesh(core_axis_name="c", subcore_axis_name="s")

    @functools.partial(
        pl.kernel,
        mesh=mesh,
        out_type=(
            jax.ShapeDtypeStruct((NB, D), jnp.float32),
            jax.ShapeDtypeStruct((NB, 128), jnp.float32),
        ),
        scratch_types=[
            pltpu.VMEM((TPW, D), jnp.float32),
            pltpu.VMEM((TPW, 128), jnp.float32),
            pltpu.VMEM((TPW, 128), jnp.float32),
            pltpu.VMEM((TPW,), jnp.int32),
            pltpu.VMEM((TPW,), jnp.int32),
            pltpu.SemaphoreType.DMA,
        ],
    )
    def sc_scatter(x_hbm, pos_hbm, wt_hbm, xs_hbm, ws_hbm,
                   xr_v, wv0_v, wv1_v, p0_v, p1_v, sem):
        w = lax.axis_index("s") * NC + lax.axis_index("c")
        pltpu.sync_copy(x_hbm.at[pl.ds(w * TPW, TPW)], xr_v)
        pltpu.sync_copy(pos_hbm.at[w], p0_v)
        pltpu.sync_copy(pos_hbm.at[NW + w], p1_v)
        pltpu.sync_copy(wt_hbm.at[w], wv0_v)
        pltpu.sync_copy(wt_hbm.at[NW + w], wv1_v)
        cps = [
            pltpu.async_copy(xr_v, xs_hbm.at[p0_v], sem),
            pltpu.async_copy(xr_v, xs_hbm.at[p1_v], sem),
            pltpu.async_copy(wv0_v, ws_hbm.at[p0_v], sem),
            pltpu.async_copy(wv1_v, ws_hbm.at[p1_v], sem),
        ]
        for cp in cps:
            cp.wait()

    @functools.partial(
        pl.kernel,
        mesh=mesh,
        out_type=jax.ShapeDtypeStruct((N, D), jnp.float32),
        scratch_types=[
            pltpu.VMEM((TPW, D), jnp.float32),
            pltpu.VMEM((TPW, D), jnp.float32),
            pltpu.VMEM((TPW,), jnp.int32),
            pltpu.VMEM((TPW,), jnp.int32),
            pltpu.SemaphoreType.DMA,
        ],
    )
    def sc_combine(ys_hbm, pos_hbm, out_hbm, r_v, o_v, p0_v, p1_v, sem):
        w = lax.axis_index("s") * NC + lax.axis_index("c")

        pltpu.sync_copy(pos_hbm.at[w], p0_v)
        pltpu.sync_copy(pos_hbm.at[NW + w], p1_v)
        cps = [
            pltpu.async_copy(ys_hbm.at[p0_v], o_v, sem),
            pltpu.async_copy(ys_hbm.at[p1_v], r_v, sem),
        ]
        for cp in cps:
            cp.wait()

        def addrow(i, _):
            for v in range(VPR):
                sl = pl.ds(v * LANES, LANES)
                o_v[i, sl] = o_v[i, sl] + r_v[i, sl]
            return 0

        lax.fori_loop(0, TPW, addrow, 0)
        pltpu.sync_copy(o_v, out_hbm.at[pl.ds(w * TPW, TPW)])

    return sc_scatter, sc_combine


def kernel(x, gate_W, gate_b, W1, b1, W2, b2):
    tw, pos, be = pl.pallas_call(
        _gate_route_body,
        out_shape=(
            jax.ShapeDtypeStruct((K * N, 128), jnp.float32),
            jax.ShapeDtypeStruct((N, K), jnp.int32),
            jax.ShapeDtypeStruct((1, G), jnp.int32),
        ),
    )(x, gate_W, gate_b.reshape(1, E))
    pos_scat = pos.T.reshape(K * NW, TPW)
    wt_scat = tw.reshape(K * NW, TPW, 128)
    be_flat = be.reshape(G)

    sc_scatter, sc_combine = _sc_kernels()
    xs, ws = sc_scatter(x, pos_scat, wt_scat)
    ys = _grouped_ffn(be_flat, xs, W1, b1, W2, b2, ws)
    return sc_combine(ys, pos_scat)


# speedup vs baseline: 1.1344x; 1.1344x over previous
import functools

import jax
import jax.numpy as jnp
from jax import lax
from jax.experimental import pallas as pl
from jax.experimental.pallas import tpu as pltpu
from jax.experimental.pallas import tpu_sc as plsc

N = 2048
D = 768
E = 8
K = 2
F = 4 * D
BLK = 128
G = (N * K) // BLK + E
NB = G * BLK

NC, NS = 2, 16
NW = NC * NS
TPW = N // NW
LANES = 16
VPR = D // LANES


def _gate_route_body(x_ref, gw_ref, gb_ref, tw_ref, pos_ref, off_ref, nblk_ref):
    xv = x_ref[...]
    logits = jnp.dot(xv, gw_ref[...], preferred_element_type=jnp.float32)
    logits = logits + gb_ref[...]
    m = jnp.max(logits, axis=1, keepdims=True)
    ex = jnp.exp(logits - m)
    p = ex / jnp.sum(ex, axis=1, keepdims=True)

    col = lax.broadcasted_iota(jnp.int32, (N, E), 1)
    m0 = jnp.max(p, axis=1, keepdims=True)
    i0 = jnp.min(jnp.where(p == m0, col, E), axis=1, keepdims=True)
    oh0 = (col == i0)
    p1 = jnp.where(oh0, -jnp.inf, p)
    m1 = jnp.max(p1, axis=1, keepdims=True)
    i1 = jnp.min(jnp.where(p1 == m1, col, E), axis=1, keepdims=True)
    oh1 = (col == i1)
    z128 = jnp.zeros((N, 128), jnp.float32)
    tw_ref[0:N, :] = m0 + z128
    tw_ref[N:2 * N, :] = m1 + z128

    ohf = (oh0 | oh1).astype(jnp.float32)
    c = ohf
    sh = 1
    while sh < N:
        c = c + jnp.concatenate([jnp.zeros((sh, E), jnp.float32), c[: N - sh]], axis=0)
        sh *= 2
    excl = c - ohf
    counts = jnp.sum(ohf, axis=0, keepdims=True)
    ci = counts.astype(jnp.int32)
    pc = ((ci + BLK - 1) // BLK) * BLK
    inc = pc
    sh = 1
    while sh < E:
        inc = inc + jnp.concatenate(
            [jnp.zeros((1, sh), jnp.int32), inc[:, : E - sh]], axis=1)
        sh *= 2
    off = (inc - pc).astype(jnp.float32)

    oh0f = oh0.astype(jnp.float32)
    oh1f = oh1.astype(jnp.float32)
    r0 = jnp.sum(excl * oh0f, axis=1, keepdims=True)
    r1 = jnp.sum(excl * oh1f, axis=1, keepdims=True)
    s0 = jnp.sum(off * oh0f, axis=1, keepdims=True)
    s1 = jnp.sum(off * oh1f, axis=1, keepdims=True)
    pos_ref[...] = jnp.concatenate([s0 + r0, s1 + r1], axis=1).astype(jnp.int32)

    off_ref[...] = inc - pc
    nblk_ref[...] = pc // BLK


TF = 4
FT = F // TF


def _ffn_body(offs_ref, nblks_ref, x_ref, w1_ref, b1_ref, w2_ref, b2_ref,
              ws_ref, y_ref):
    e = pl.program_id(0)
    t = pl.program_id(1)
    base = offs_ref[e]
    nb = nblks_ref[e]
    w1b = w1_ref[0].astype(jnp.bfloat16)
    w2b = w2_ref[0].astype(jnp.bfloat16)
    b1v = b1_ref[0]
    b2v = b2_ref[0]

    def blk(j, _):
        st = pl.multiple_of(base + j * BLK, BLK)
        xb = x_ref[pl.ds(st, BLK), :].astype(jnp.bfloat16)
        h = jnp.dot(xb, w1b, preferred_element_type=jnp.float32)
        h = jnp.maximum(h + b1v, 0.0).astype(jnp.bfloat16)
        y = jnp.dot(h, w2b, preferred_element_type=jnp.float32)

        @pl.when(t == 0)
        def _():
            y_ref[pl.ds(st, BLK), :] = y + b2v

        @pl.when(jnp.logical_and(t > 0, t < TF - 1))
        def _():
            y_ref[pl.ds(st, BLK), :] = y_ref[pl.ds(st, BLK), :] + y

        @pl.when(t == TF - 1)
        def _():
            acc = y_ref[pl.ds(st, BLK), :] + y
            y_ref[pl.ds(st, BLK), :] = acc * ws_ref[pl.ds(st, BLK), :1]

        return 0

    lax.fori_loop(0, nb, blk, 0)


def _grouped_ffn(offs, nblks, xs, W1, b1, W2, b2, ws):
    grid_spec = pltpu.PrefetchScalarGridSpec(
        num_scalar_prefetch=2,
        grid=(E, TF),
        in_specs=[
            pl.BlockSpec((NB, D), lambda e, t, o, nb: (0, 0)),
            pl.BlockSpec((1, D, FT), lambda e, t, o, nb: (e, 0, t)),
            pl.BlockSpec((1, 1, FT), lambda e, t, o, nb: (e, 0, t)),
            pl.BlockSpec((1, FT, D), lambda e, t, o, nb: (e, t, 0)),
            pl.BlockSpec((1, 1, D), lambda e, t, o, nb: (e, 0, 0)),
            pl.BlockSpec((NB, 128), lambda e, t, o, nb: (0, 0)),
        ],
        out_specs=pl.BlockSpec((NB, D), lambda e, t, o, nb: (0, 0)),
    )
    return pl.pallas_call(
        _ffn_body,
        grid_spec=grid_spec,
        out_shape=jax.ShapeDtypeStruct((NB, D), jnp.float32),
        compiler_params=pltpu.CompilerParams(
            dimension_semantics=("arbitrary", "arbitrary")),
    )(offs, nblks, xs, W1, b1.reshape(E, 1, F), W2, b2.reshape(E, 1, D), ws)


@functools.cache
def _sc_kernels():
    mesh = plsc.VectorSubcoreMesh(core_axis_name="c", subcore_axis_name="s")

    @functools.partial(
        pl.kernel,
        mesh=mesh,
        out_type=(
            jax.ShapeDtypeStruct((NB, D), jnp.float32),
            jax.ShapeDtypeStruct((NB, 128), jnp.float32),
        ),
        scratch_types=[
            pltpu.VMEM((TPW, D), jnp.float32),
            pltpu.VMEM((TPW, 128), jnp.float32),
            pltpu.VMEM((TPW, 128), jnp.float32),
            pltpu.VMEM((TPW,), jnp.int32),
            pltpu.VMEM((TPW,), jnp.int32),
            pltpu.SemaphoreType.DMA,
        ],
    )
    def sc_scatter(x_hbm, pos_hbm, wt_hbm, xs_hbm, ws_hbm,
                   xr_v, wv0_v, wv1_v, p0_v, p1_v, _sem):
        w = lax.axis_index("s") * NC + lax.axis_index("c")
        pltpu.sync_copy(x_hbm.at[pl.ds(w * TPW, TPW)], xr_v)
        pltpu.sync_copy(pos_hbm.at[w], p0_v)
        pltpu.sync_copy(pos_hbm.at[NW + w], p1_v)
        pltpu.sync_copy(wt_hbm.at[w], wv0_v)
        pltpu.sync_copy(wt_hbm.at[NW + w], wv1_v)
        pltpu.sync_copy(xr_v, xs_hbm.at[p0_v])
        pltpu.sync_copy(xr_v, xs_hbm.at[p1_v])
        pltpu.sync_copy(wv0_v, ws_hbm.at[p0_v])
        pltpu.sync_copy(wv1_v, ws_hbm.at[p1_v])

    @functools.partial(
        pl.kernel,
        mesh=mesh,
        out_type=jax.ShapeDtypeStruct((N, D), jnp.float32),
        scratch_types=[
            pltpu.VMEM((TPW, D), jnp.float32),
            pltpu.VMEM((TPW, D), jnp.float32),
            pltpu.VMEM((TPW,), jnp.int32),
            pltpu.VMEM((TPW,), jnp.int32),
            pltpu.SemaphoreType.DMA,
        ],
    )
    def sc_combine(ys_hbm, pos_hbm, out_hbm, r_v, o_v, p0_v, p1_v, sem):
        w = lax.axis_index("s") * NC + lax.axis_index("c")

        pltpu.sync_copy(pos_hbm.at[w], p0_v)
        pltpu.sync_copy(pos_hbm.at[NW + w], p1_v)
        cps = [
            pltpu.async_copy(ys_hbm.at[p0_v], o_v, sem),
            pltpu.async_copy(ys_hbm.at[p1_v], r_v, sem),
        ]
        for cp in cps:
            cp.wait()

        def addrow(i, _):
            for v in range(VPR):
                sl = pl.ds(v * LANES, LANES)
                o_v[i, sl] = o_v[i, sl] + r_v[i, sl]
            return 0

        lax.fori_loop(0, TPW, addrow, 0)
        pltpu.sync_copy(o_v, out_hbm.at[pl.ds(w * TPW, TPW)])

    return sc_scatter, sc_combine


def kernel(x, gate_W, gate_b, W1, b1, W2, b2):
    tw, pos, offs, nblks = pl.pallas_call(
        _gate_route_body,
        out_shape=(
            jax.ShapeDtypeStruct((K * N, 128), jnp.float32),
            jax.ShapeDtypeStruct((N, K), jnp.int32),
            jax.ShapeDtypeStruct((1, E), jnp.int32),
            jax.ShapeDtypeStruct((1, E), jnp.int32),
        ),
    )(x, gate_W, gate_b.reshape(1, E))
    pos_scat = pos.T.reshape(K * NW, TPW)
    wt_scat = tw.reshape(K * NW, TPW, 128)

    sc_scatter, sc_combine = _sc_kernels()
    xs, ws = sc_scatter(x, pos_scat, wt_scat)
    ys = _grouped_ffn(offs.reshape(E), nblks.reshape(E), xs, W1, b1, W2, b2, ws)
    return sc_combine(ys, pos_scat)


# speedup vs baseline: 1.2133x; 1.0695x over previous
import functools

import jax
import jax.numpy as jnp
from jax import lax
from jax.experimental import pallas as pl
from jax.experimental.pallas import tpu as pltpu
from jax.experimental.pallas import tpu_sc as plsc

N = 2048
D = 768
E = 8
K = 2
F = 4 * D
BLK = 128
G = (N * K) // BLK + E - 1
NB = G * BLK

NC, NS = 2, 16
NW = NC * NS
TPW = N // NW
LANES = 16
VPR = D // LANES


def _gate_route_body(x_ref, gw_ref, gb_ref, tw_ref, pos_ref,
                     off_ref, nblk_ref):
    xv = x_ref[...]
    logits = jnp.dot(xv, gw_ref[...], preferred_element_type=jnp.float32)
    logits = logits + gb_ref[...]
    m = jnp.max(logits, axis=1, keepdims=True)
    ex = jnp.exp(logits - m)
    p = ex / jnp.sum(ex, axis=1, keepdims=True)

    col = lax.broadcasted_iota(jnp.int32, (N, E), 1)
    m0 = jnp.max(p, axis=1, keepdims=True)
    i0 = jnp.min(jnp.where(p == m0, col, E), axis=1, keepdims=True)
    oh0 = (col == i0)
    p1 = jnp.where(oh0, -jnp.inf, p)
    m1 = jnp.max(p1, axis=1, keepdims=True)
    i1 = jnp.min(jnp.where(p1 == m1, col, E), axis=1, keepdims=True)
    oh1 = (col == i1)
    z128 = jnp.zeros((N, 128), jnp.float32)
    tw_ref[0:N, :] = m0 + z128
    tw_ref[N:2 * N, :] = m1 + z128

    ohf = (oh0 | oh1).astype(jnp.float32)
    c = ohf
    sh = 1
    while sh < N:
        c = c + jnp.concatenate([jnp.zeros((sh, E), jnp.float32), c[: N - sh]], axis=0)
        sh *= 2
    excl = c - ohf
    counts = jnp.sum(ohf, axis=0, keepdims=True)
    ci = counts.astype(jnp.int32)
    pc = ((ci + BLK - 1) // BLK) * BLK
    inc = pc
    sh = 1
    while sh < E:
        inc = inc + jnp.concatenate(
            [jnp.zeros((1, sh), jnp.int32), inc[:, : E - sh]], axis=1)
        sh *= 2
    off = (inc - pc).astype(jnp.float32)

    oh0f = oh0.astype(jnp.float32)
    oh1f = oh1.astype(jnp.float32)
    r0 = jnp.sum(excl * oh0f, axis=1, keepdims=True)
    r1 = jnp.sum(excl * oh1f, axis=1, keepdims=True)
    s0 = jnp.sum(off * oh0f, axis=1, keepdims=True)
    s1 = jnp.sum(off * oh1f, axis=1, keepdims=True)
    pos_ref[...] = jnp.concatenate([s0 + r0, s1 + r1], axis=1).astype(jnp.int32)

    off_ref[...] = inc - pc
    nblk_ref[...] = pc // BLK


def _ffn_body(offs_ref, nblks_ref, x_hbm, w1_ref, b1_ref, w2_ref, b2_ref,
              ws_hbm, y_hbm, x_scr, ws_scr, y_scr, sx, sw, sy):
    e = pl.program_id(0)
    base = offs_ref[e]
    nb = nblks_ref[e]
    b1v = b1_ref[0]
    b2v = b2_ref[0]

    def rowds(j):
        return pl.ds(pl.multiple_of(base + j * BLK, BLK), BLK)

    def xcp(j, s):
        return pltpu.make_async_copy(x_hbm.at[rowds(j), :], x_scr.at[s], sx.at[s])

    def wcp(j, s):
        return pltpu.make_async_copy(ws_hbm.at[rowds(j), :], ws_scr.at[s], sw.at[s])

    def ycp(j, s):
        return pltpu.make_async_copy(y_scr.at[s], y_hbm.at[rowds(j), :], sy.at[s])

    @pl.when(nb > 0)
    def _():
        xcp(0, 0).start()
        wcp(0, 0).start()

    def blk(j, _):
        s = lax.rem(j, 2)
        ns = 1 - s

        @pl.when(j + 1 < nb)
        def _():
            xcp(j + 1, ns).start()
            wcp(j + 1, ns).start()

        xcp(j, s).wait()
        wcp(j, s).wait()
        xv = x_scr[s]
        h = jnp.dot(xv, w1_ref[0], preferred_element_type=jnp.float32)
        h = jnp.maximum(h + b1v, 0.0)
        y = jnp.dot(h, w2_ref[0], preferred_element_type=jnp.float32) + b2v

        @pl.when(j >= 2)
        def _():
            ycp(j - 2, s).wait()

        y_scr[s] = y * ws_scr[s, :, :1]
        ycp(j, s).start()
        return 0

    lax.fori_loop(0, nb, blk, 0)

    @pl.when(nb >= 2)
    def _():
        ycp(nb - 2, lax.rem(nb - 2, 2)).wait()

    @pl.when(nb >= 1)
    def _():
        ycp(nb - 1, lax.rem(nb - 1, 2)).wait()


def _grouped_ffn(offs, nblks, xs, W1, b1, W2, b2, ws):
    grid_spec = pltpu.PrefetchScalarGridSpec(
        num_scalar_prefetch=2,
        grid=(E,),
        in_specs=[
            pl.BlockSpec(memory_space=pl.ANY),
            pl.BlockSpec((1, D, F), lambda e, o, nb: (e, 0, 0)),
            pl.BlockSpec((1, 1, F), lambda e, o, nb: (e, 0, 0)),
            pl.BlockSpec((1, F, D), lambda e, o, nb: (e, 0, 0)),
            pl.BlockSpec((1, 1, D), lambda e, o, nb: (e, 0, 0)),
            pl.BlockSpec(memory_space=pl.ANY),
        ],
        out_specs=pl.BlockSpec(memory_space=pl.ANY),
        scratch_shapes=[
            pltpu.VMEM((2, BLK, D), jnp.float32),
            pltpu.VMEM((2, BLK, 128), jnp.float32),
            pltpu.VMEM((2, BLK, D), jnp.float32),
            pltpu.SemaphoreType.DMA((2,)),
            pltpu.SemaphoreType.DMA((2,)),
            pltpu.SemaphoreType.DMA((2,)),
        ],
    )
    return pl.pallas_call(
        _ffn_body,
        grid_spec=grid_spec,
        out_shape=jax.ShapeDtypeStruct((NB, D), jnp.float32),
        compiler_params=pltpu.CompilerParams(
            dimension_semantics=("arbitrary",)),
    )(offs, nblks, xs, W1, b1.reshape(E, 1, F), W2, b2.reshape(E, 1, D), ws)


@functools.cache
def _sc_kernels():
    mesh = plsc.VectorSubcoreMesh(core_axis_name="c", subcore_axis_name="s")

    @functools.partial(
        pl.kernel,
        mesh=mesh,
        out_type=(
            jax.ShapeDtypeStruct((NB, D), jnp.float32),
            jax.ShapeDtypeStruct((NB, 128), jnp.float32),
        ),
        scratch_types=[
            pltpu.VMEM((TPW, D), jnp.float32),
            pltpu.VMEM((TPW, 128), jnp.float32),
            pltpu.VMEM((TPW, 128), jnp.float32),
            pltpu.VMEM((TPW,), jnp.int32),
            pltpu.VMEM((TPW,), jnp.int32),
            pltpu.SemaphoreType.DMA,
        ],
    )
    def sc_scatter(x_hbm, pos_hbm, wt_hbm, xs_hbm, ws_hbm,
                   xr_v, wv0_v, wv1_v, p0_v, p1_v, _sem):
        w = lax.axis_index("s") * NC + lax.axis_index("c")
        pltpu.sync_copy(x_hbm.at[pl.ds(w * TPW, TPW)], xr_v)
        pltpu.sync_copy(pos_hbm.at[w], p0_v)
        pltpu.sync_copy(pos_hbm.at[NW + w], p1_v)
        pltpu.sync_copy(wt_hbm.at[w], wv0_v)
        pltpu.sync_copy(wt_hbm.at[NW + w], wv1_v)
        pltpu.sync_copy(xr_v, xs_hbm.at[p0_v])
        pltpu.sync_copy(xr_v, xs_hbm.at[p1_v])
        pltpu.sync_copy(wv0_v, ws_hbm.at[p0_v])
        pltpu.sync_copy(wv1_v, ws_hbm.at[p1_v])

    @functools.partial(
        pl.kernel,
        mesh=mesh,
        out_type=jax.ShapeDtypeStruct((N, D), jnp.float32),
        scratch_types=[
            pltpu.VMEM((TPW, D), jnp.float32),
            pltpu.VMEM((TPW, D), jnp.float32),
            pltpu.VMEM((TPW,), jnp.int32),
            pltpu.VMEM((TPW,), jnp.int32),
            pltpu.SemaphoreType.DMA,
        ],
    )
    def sc_combine(ys_hbm, pos_hbm, out_hbm, r_v, o_v, p0_v, p1_v, sem):
        w = lax.axis_index("s") * NC + lax.axis_index("c")

        pltpu.sync_copy(pos_hbm.at[w], p0_v)
        pltpu.sync_copy(pos_hbm.at[NW + w], p1_v)
        cps = [
            pltpu.async_copy(ys_hbm.at[p0_v], o_v, sem),
            pltpu.async_copy(ys_hbm.at[p1_v], r_v, sem),
        ]
        for cp in cps:
            cp.wait()

        def addrow(i, _):
            for v in range(VPR):
                sl = pl.ds(v * LANES, LANES)
                o_v[i, sl] = o_v[i, sl] + r_v[i, sl]
            return 0

        lax.fori_loop(0, TPW, addrow, 0)
        pltpu.sync_copy(o_v, out_hbm.at[pl.ds(w * TPW, TPW)])

    return sc_scatter, sc_combine


def kernel(x, gate_W, gate_b, W1, b1, W2, b2):
    tw, pos, offs, nblks = pl.pallas_call(
        _gate_route_body,
        out_shape=(
            jax.ShapeDtypeStruct((K * N, 128), jnp.float32),
            jax.ShapeDtypeStruct((N, K), jnp.int32),
            jax.ShapeDtypeStruct((1, E), jnp.int32),
            jax.ShapeDtypeStruct((1, E), jnp.int32),
        ),
    )(x, gate_W, gate_b.reshape(1, E))
    pos_scat = pos.T.reshape(K * NW, TPW)
    wt_scat = tw.reshape(K * NW, TPW, 128)

    sc_scatter, sc_combine = _sc_kernels()
    xs, ws = sc_scatter(x, pos_scat, wt_scat)
    ys = _grouped_ffn(offs.reshape(E), nblks.reshape(E), xs, W1, b1, W2, b2, ws)
    return sc_combine(ys, pos_scat)
